# Initial kernel scaffold; baseline (speedup 1.0000x reference)
import jax
import jax.numpy as jnp
from jax.experimental import pallas as pl


def kernel(h, edge_index, W0, W1, W2, W3, W4, b0, b1, b2, b3, b4):
    raise NotImplementedError("write your pallas kernel here")


# speedup vs baseline: 7.1759x; 7.1759x over previous
import functools

import jax
import jax.numpy as jnp
from jax import lax
from jax.experimental import pallas as pl
from jax.experimental.pallas import tpu as pltpu
from jax.experimental.pallas import tpu_sc as plsc

N_NODES = 10000
D = 128
N_EDGES = 320000

NC = 2
NS = 16
NW = NC * NS
E_PER_W = N_EDGES // NW
CHUNK = 80
NCHUNK = E_PER_W // CHUNK
N_PAD = 10240
ROWS_PER_TILE = N_PAD // NS
ZROWS = 16


def _sc_agg_body(h_hbm, src_hbm, dst_hbm, out_hbm,
                 src_idx, dst_idx, rows0, zbuf, agg_sh, sem0):
    c = lax.axis_index("c")
    s = lax.axis_index("s")
    wid = s * NC + c

    pltpu.sync_copy(src_hbm.at[wid], src_idx)
    pltpu.sync_copy(dst_hbm.at[wid], dst_idx)

    def _z(i, _):
        zbuf[i // 8, pl.ds((i % 8) * 16, 16)] = jnp.zeros((16,), jnp.float32)
        return _
    lax.fori_loop(0, ZROWS * 8, _z, None)

    def _zcopy(j, _):
        pltpu.sync_copy(zbuf, agg_sh.at[pl.ds(s * ROWS_PER_TILE + j * ZROWS, ZROWS)])
        return _
    lax.fori_loop(0, ROWS_PER_TILE // ZROWS, _zcopy, None)

    plsc.subcore_barrier()

    def _edges(i, _):
        pltpu.async_copy(h_hbm.at[src_idx.at[i]], rows0, sem0).wait()
        pltpu.sync_copy(rows0, agg_sh.at[dst_idx.at[i]], add=True)
        return _

    lax.fori_loop(0, NCHUNK, _edges, None)

    plsc.subcore_barrier()

    pltpu.sync_copy(agg_sh.at[pl.ds(s * ROWS_PER_TILE, ROWS_PER_TILE)],
                    out_hbm.at[c, pl.ds(s * ROWS_PER_TILE, ROWS_PER_TILE)])


_sc_agg = pl.kernel(
    _sc_agg_body,
    out_type=jax.ShapeDtypeStruct((NC, N_PAD, D), jnp.float32),
    mesh=plsc.VectorSubcoreMesh(core_axis_name="c", subcore_axis_name="s"),
    scratch_types=[
        pltpu.VMEM((NCHUNK, CHUNK), jnp.int32),
        pltpu.VMEM((NCHUNK, CHUNK), jnp.int32),
        pltpu.VMEM((CHUNK, D), jnp.float32),
        pltpu.VMEM((ZROWS, D), jnp.float32),
        pltpu.VMEM_SHARED((N_PAD, D), jnp.float32),
        pltpu.SemaphoreType.DMA,
    ],
)


def _mm_body(h_ref, a_ref, w_ref, b_ref, o_ref):
    x = h_ref[...] + a_ref[0] + a_ref[1]
    o_ref[...] = jnp.dot(x, w_ref[...], preferred_element_type=jnp.float32) + b_ref[...]


_ROWS_BLK = 1000


def _mm(h, agg, w, b):
    return pl.pallas_call(
        _mm_body,
        grid=(N_NODES // _ROWS_BLK,),
        in_specs=[
            pl.BlockSpec((_ROWS_BLK, D), lambda i: (i, 0)),
            pl.BlockSpec((NC, _ROWS_BLK, D), lambda i: (0, i, 0)),
            pl.BlockSpec((D, D), lambda i: (0, 0)),
            pl.BlockSpec((1, D), lambda i: (0, 0)),
        ],
        out_specs=pl.BlockSpec((_ROWS_BLK, D), lambda i: (i, 0)),
        out_shape=jax.ShapeDtypeStruct((N_NODES, D), jnp.float32),
    )(h, agg, w, b.reshape(1, D))


def kernel(h, edge_index, W0, W1, W2, W3, W4, b0, b1, b2, b3, b4):
    Ws = [W0, W1, W2, W3, W4]
    bs = [b0, b1, b2, b3, b4]
    src3 = edge_index[0].reshape(NW, NCHUNK, CHUNK)
    dst3 = edge_index[1].reshape(NW, NCHUNK, CHUNK)
    for i in range(5):
        agg = _sc_agg(h, src3, dst3)
        h = _mm(h, agg, Ws[i], bs[i])
    return h
